# Initial kernel scaffold; baseline (speedup 1.0000x reference)
import jax
import jax.numpy as jnp
from jax.experimental import pallas as pl


def kernel(input_ids, table):
    raise NotImplementedError("write your pallas kernel here")


# speedup vs baseline: 3.2721x; 3.2721x over previous
import functools

import jax
import jax.numpy as jnp
from jax import lax
from jax.experimental import pallas as pl
from jax.experimental.pallas import tpu as pltpu
from jax.experimental.pallas import tpu_sc as plsc

NUM_EMBEDDINGS = 100000
EMBEDDING_DIM = 128
BATCH = 4096
SEQ = 50
N_TOKENS = BATCH * SEQ

_INFO = plsc.get_sparse_core_info()
_NW = _INFO.num_cores * _INFO.num_subcores
_PER_W = N_TOKENS // _NW
_CHUNK = 640
_NSTEP = _PER_W // _CHUNK


def _sc_gather(ids_flat, table):
  mesh = plsc.VectorSubcoreMesh(core_axis_name="c", subcore_axis_name="s")

  @functools.partial(
      pl.kernel,
      out_type=jax.ShapeDtypeStruct((N_TOKENS, EMBEDDING_DIM), jnp.float32),
      mesh=mesh,
      scratch_types=[
          pltpu.VMEM((_CHUNK,), jnp.int32),
          pltpu.VMEM((_CHUNK, EMBEDDING_DIM), jnp.float32),
          pltpu.SemaphoreType.DMA,
      ],
  )
  def body(ids_hbm, table_hbm, out_hbm, idx_v, rows_v, sem):
    wid = lax.axis_index("s") * _INFO.num_cores + lax.axis_index("c")
    base = wid * _PER_W
    for g in range(_NSTEP):
      off = base + g * _CHUNK
      pltpu.sync_copy(ids_hbm.at[pl.ds(off, _CHUNK)], idx_v)
      pltpu.async_copy(table_hbm.at[idx_v], rows_v, sem).wait()
      pltpu.sync_copy(rows_v, out_hbm.at[pl.ds(off, _CHUNK)])

  return body(ids_flat, table)


def kernel(input_ids, table):
  ids_flat = input_ids.reshape(N_TOKENS)
  out = _sc_gather(ids_flat, table)
  return out.reshape(BATCH, SEQ, EMBEDDING_DIM)


# speedup vs baseline: 3.3308x; 1.0179x over previous
import functools

import jax
import jax.numpy as jnp
from jax import lax
from jax.experimental import pallas as pl
from jax.experimental.pallas import tpu as pltpu
from jax.experimental.pallas import tpu_sc as plsc

NUM_EMBEDDINGS = 100000
EMBEDDING_DIM = 128
BATCH = 4096
SEQ = 50
N_TOKENS = BATCH * SEQ

_INFO = plsc.get_sparse_core_info()
_NW = _INFO.num_cores * _INFO.num_subcores
_PER_W = N_TOKENS // _NW
_CHUNK = 400
_NSTEP = _PER_W // _CHUNK


def _sc_gather(ids_flat, table):
  mesh = plsc.VectorSubcoreMesh(core_axis_name="c", subcore_axis_name="s")

  @functools.partial(
      pl.kernel,
      out_type=jax.ShapeDtypeStruct((N_TOKENS, EMBEDDING_DIM), jnp.float32),
      mesh=mesh,
      scratch_types=[
          pltpu.VMEM((_CHUNK,), jnp.int32),
          pltpu.VMEM((_CHUNK,), jnp.int32),
          pltpu.VMEM((_CHUNK, EMBEDDING_DIM), jnp.float32),
          pltpu.VMEM((_CHUNK, EMBEDDING_DIM), jnp.float32),
          pltpu.SemaphoreType.DMA,
          pltpu.SemaphoreType.DMA,
          pltpu.SemaphoreType.DMA,
          pltpu.SemaphoreType.DMA,
      ],
  )
  def body(ids_hbm, table_hbm, out_hbm, idx0, idx1, rows0, rows1,
           gsem0, gsem1, ssem0, ssem1):
    wid = lax.axis_index("s") * _INFO.num_cores + lax.axis_index("c")
    base = wid * _PER_W
    idx_v = (idx0, idx1)
    rows_v = (rows0, rows1)
    gsem = (gsem0, gsem1)
    ssem = (ssem0, ssem1)

    gathers = [None] * _NSTEP
    stores = [None] * _NSTEP

    def start_gather(g):
      off = base + g * _CHUNK
      b = g % 2
      pltpu.sync_copy(ids_hbm.at[pl.ds(off, _CHUNK)], idx_v[b])
      gathers[g] = pltpu.async_copy(table_hbm.at[idx_v[b]], rows_v[b], gsem[b])

    start_gather(0)
    for g in range(_NSTEP):
      b = g % 2
      if g + 1 < _NSTEP:
        if g >= 1:
          stores[g - 1].wait()
        start_gather(g + 1)
      gathers[g].wait()
      stores[g] = pltpu.async_copy(
          rows_v[b], out_hbm.at[pl.ds(base + g * _CHUNK, _CHUNK)], ssem[b])
    stores[_NSTEP - 2].wait()
    stores[_NSTEP - 1].wait()

  return body(ids_flat, table)


def kernel(input_ids, table):
  ids_flat = input_ids.reshape(N_TOKENS)
  out = _sc_gather(ids_flat, table)
  return out.reshape(BATCH, SEQ, EMBEDDING_DIM)


# speedup vs baseline: 5.8104x; 1.7444x over previous
import functools

import jax
import jax.numpy as jnp
from jax import lax
from jax.experimental import pallas as pl
from jax.experimental.pallas import tpu as pltpu
from jax.experimental.pallas import tpu_sc as plsc

NUM_EMBEDDINGS = 100000
EMBEDDING_DIM = 128
BATCH = 4096
SEQ = 50
N_TOKENS = BATCH * SEQ

_INFO = plsc.get_sparse_core_info()
_NW = _INFO.num_cores * _INFO.num_subcores
_B_PER_W = BATCH // _NW
_CB = 8
_CHUNK = _CB * SEQ
_NSTEP = _B_PER_W // _CB


def _sc_gather(ids_flat, table):
  mesh = plsc.VectorSubcoreMesh(core_axis_name="c", subcore_axis_name="s")

  @functools.partial(
      pl.kernel,
      out_type=jax.ShapeDtypeStruct((BATCH, SEQ, EMBEDDING_DIM), jnp.float32),
      mesh=mesh,
      scratch_types=[
          pltpu.VMEM((_CHUNK,), jnp.int32),
          pltpu.VMEM((_CHUNK,), jnp.int32),
          pltpu.VMEM((_CHUNK, EMBEDDING_DIM), jnp.float32),
          pltpu.VMEM((_CHUNK, EMBEDDING_DIM), jnp.float32),
          pltpu.SemaphoreType.DMA,
          pltpu.SemaphoreType.DMA,
          pltpu.SemaphoreType.DMA,
          pltpu.SemaphoreType.DMA,
      ],
  )
  def body(ids_hbm, table_hbm, out_hbm, idx0, idx1, rows0, rows1,
           gsem0, gsem1, ssem0, ssem1):
    wid = lax.axis_index("s") * _INFO.num_cores + lax.axis_index("c")
    batch0 = wid * _B_PER_W
    idx_v = (idx0, idx1)
    rows_v = (rows0, rows1)
    gsem = (gsem0, gsem1)
    ssem = (ssem0, ssem1)

    gathers = [None] * _NSTEP
    stores = [None] * _NSTEP

    def start_gather(g):
      b = g % 2
      off = (batch0 + g * _CB) * SEQ
      pltpu.sync_copy(ids_hbm.at[pl.ds(off, _CHUNK)], idx_v[b])
      gathers[g] = pltpu.async_copy(table_hbm.at[idx_v[b]], rows_v[b], gsem[b])

    def start_stores(g):
      b = g % 2
      copies = []
      for i in range(_CB):
        copies.append(pltpu.async_copy(
            rows_v[b].at[pl.ds(i * SEQ, SEQ)],
            out_hbm.at[batch0 + g * _CB + i],
            ssem[b]))
      return copies

    start_gather(0)
    for g in range(_NSTEP):
      if g + 1 < _NSTEP:
        if g >= 1:
          for c in stores[g - 1]:
            c.wait()
        start_gather(g + 1)
      gathers[g].wait()
      stores[g] = start_stores(g)
    for g in (_NSTEP - 2, _NSTEP - 1):
      for c in stores[g]:
        c.wait()

  return body(ids_flat, table)


def kernel(input_ids, table):
  ids_flat = input_ids.reshape(N_TOKENS)
  return _sc_gather(ids_flat, table)
